# baseline (speedup 1.0000x reference)
import functools

import jax
import jax.numpy as jnp
from jax import lax
from jax.experimental import pallas as pl
from jax.experimental.pallas import tpu as pltpu
from jax.experimental.pallas import tpu_sc as plsc

N_NODE = 10000
E = 160000
HID = 128

NSUB = 16
PAD_N = 10240
DUMMY = PAD_N - 1
CHUNK = 128
EPAD = 163840
NCHUNK = EPAD // (NSUB * CHUNK)
ROWS_PER_SUB = PAD_N // NSUB

_sc_mesh = plsc.VectorSubcoreMesh(core_axis_name="c", subcore_axis_name="s")


def _fill_rows(ref, nrows, ncols16, value):
    @pl.loop(0, nrows)
    def _(i):
        for k in range(ncols16):
            ref[i, pl.ds(k * 16, 16)] = jnp.full((16,), value, jnp.float32)


HCHUNK = NCHUNK // 2


@functools.partial(
    pl.kernel,
    out_type=jax.ShapeDtypeStruct((2, PAD_N, HID), jnp.float32),
    mesh=_sc_mesh,
    scratch_types=[
        pltpu.VMEM((HCHUNK, CHUNK), jnp.int32),
        pltpu.VMEM((HCHUNK, CHUNK), jnp.int32),
        pltpu.VMEM((CHUNK, HID), jnp.float32),
        pltpu.VMEM((CHUNK, HID), jnp.float32),
        pltpu.VMEM_SHARED((PAD_N, HID), jnp.float32),
        pltpu.SemaphoreType.DMA,
        pltpu.SemaphoreType.DMA,
    ],
)
def _sc_conv(feat, e_src, e_dst, agg,
             idx_s, idx_d, rows0, rows1, acc, sem0, sem1):
    c = lax.axis_index("c")
    s = lax.axis_index("s")
    base = s * ROWS_PER_SUB
    _fill_rows(rows0, CHUNK, HID // 16, 0.0)
    for t in range(ROWS_PER_SUB // CHUNK):
        pltpu.sync_copy(rows0, acc.at[pl.ds(base + t * CHUNK, CHUNK)])
    plsc.subcore_barrier()

    for h in range(2):
        pltpu.sync_copy(e_src.at[c, s, pl.ds(h * HCHUNK, HCHUNK)], idx_s)
        pltpu.sync_copy(e_dst.at[c, s, pl.ds(h * HCHUNK, HCHUNK)], idx_d)
        pltpu.async_copy(feat.at[idx_s.at[0]], rows0, sem0)

        @pl.loop(0, HCHUNK // 2 - 1)
        def _(t):
            j0 = 2 * t
            pltpu.async_copy(feat.at[idx_s.at[j0 + 1]], rows1, sem1)
            pltpu.make_async_copy(feat.at[idx_s.at[j0]], rows0, sem0).wait()
            pltpu.sync_copy(rows0, acc.at[idx_d.at[j0]], add=True)
            pltpu.async_copy(feat.at[idx_s.at[j0 + 2]], rows0, sem0)
            pltpu.make_async_copy(feat.at[idx_s.at[j0 + 1]], rows1, sem1).wait()
            pltpu.sync_copy(rows1, acc.at[idx_d.at[j0 + 1]], add=True)

        jl = HCHUNK - 2
        pltpu.async_copy(feat.at[idx_s.at[jl + 1]], rows1, sem1)
        pltpu.make_async_copy(feat.at[idx_s.at[jl]], rows0, sem0).wait()
        pltpu.sync_copy(rows0, acc.at[idx_d.at[jl]], add=True)
        pltpu.make_async_copy(feat.at[idx_s.at[jl + 1]], rows1, sem1).wait()
        pltpu.sync_copy(rows1, acc.at[idx_d.at[jl + 1]], add=True)

    plsc.subcore_barrier()
    pltpu.sync_copy(acc.at[pl.ds(base, ROWS_PER_SUB)], agg.at[c, pl.ds(base, ROWS_PER_SUB)])


_RB = 1000


def _norm_col(hist_blk):
    return lax.rsqrt(jnp.maximum(hist_blk[:, 0:1], 1.0))


def _dot(a, b):
    return jax.lax.dot_general(a, b, (((1,), (0,)), ((), ())),
                               preferred_element_type=jnp.float32,
                               precision=jax.lax.Precision.HIGHEST)


def _tc_pre_body(frame_r, fe_r, histAs_r, histBs_r, frW_r, frb_r, feW_r, feb_r,
                 WA_r, WB_r, frame_p_r, fe_p_r, featA1_r, featB1_r):
    frame_p = _dot(frame_r[...], frW_r[...]) + frb_r[...]
    fe_p = _dot(fe_r[...], feW_r[...]) + feb_r[...]
    frame_p_r[...] = frame_p
    fe_p_r[...] = fe_p
    featA1_r[...] = _dot(fe_p * _norm_col(histAs_r[...]), WA_r[...])
    featB1_r[...] = _dot(frame_p * _norm_col(histBs_r[...]), WB_r[...])


def _tc_mid_body(aggA1_r, aggB1_r, histAd_r, histBd_r, histAs_r, histBs_r,
                 bA_r, bB_r, WA_r, WB_r, featA2_r, featB2_r):
    h_frame = jnp.maximum(aggA1_r[...] * _norm_col(histAd_r[...]) + bA_r[...], 0.0)
    h_fe = jnp.maximum(aggB1_r[...] * _norm_col(histBd_r[...]) + bB_r[...], 0.0)
    featA2_r[...] = _dot(h_fe * _norm_col(histAs_r[...]), WA_r[...])
    featB2_r[...] = _dot(h_frame * _norm_col(histBs_r[...]), WB_r[...])


def _tc_post_body(aggA2_r, aggB2_r, histAd_r, histBd_r, bA_r, bB_r,
                  frame_p_r, fe_p_r, frfcW_r, frfcb_r, fefcW_r, fefcb_r,
                  fr_h_r, fe_h_r, h2_frame_r, h2_fe_r):
    h2_frame = aggA2_r[...] * _norm_col(histAd_r[...]) + bA_r[...] + frame_p_r[...]
    h2_fe = aggB2_r[...] * _norm_col(histBd_r[...]) + bB_r[...] + fe_p_r[...]
    h2_frame_r[...] = h2_frame
    h2_fe_r[...] = h2_fe
    fr_h_r[...] = _dot(h2_frame, frfcW_r[...]) + frfcb_r[...]
    fe_h_r[...] = _dot(h2_fe, fefcW_r[...]) + fefcb_r[...]


def _rows_spec(cols):
    return pl.BlockSpec((_RB, cols), lambda i: (i, 0))


def _full_spec(r, cols):
    return pl.BlockSpec((r, cols), lambda i: (0, 0))


def _tc_pre(frame, fe, histAs, histBs, frW, frb, feW, feb, WA, WB):
    return pl.pallas_call(
        _tc_pre_body,
        grid=(N_NODE // _RB,),
        in_specs=[_rows_spec(64), _rows_spec(32), _rows_spec(HID), _rows_spec(HID),
                  _full_spec(64, HID), _full_spec(1, HID),
                  _full_spec(32, HID), _full_spec(1, HID),
                  _full_spec(HID, HID), _full_spec(HID, HID)],
        out_specs=[_rows_spec(HID)] * 4,
        out_shape=[jax.ShapeDtypeStruct((N_NODE, HID), jnp.float32)] * 4,
    )(frame, fe, histAs, histBs, frW, frb, feW, feb, WA, WB)


def _tc_mid(aggA1, aggB1, histAd, histBd, histAs, histBs, bA, bB, WA, WB):
    return pl.pallas_call(
        _tc_mid_body,
        grid=(N_NODE // _RB,),
        in_specs=[_rows_spec(HID), _rows_spec(HID),
                  _rows_spec(HID), _rows_spec(HID), _rows_spec(HID), _rows_spec(HID),
                  _full_spec(1, HID), _full_spec(1, HID),
                  _full_spec(HID, HID), _full_spec(HID, HID)],
        out_specs=[_rows_spec(HID)] * 2,
        out_shape=[jax.ShapeDtypeStruct((N_NODE, HID), jnp.float32)] * 2,
    )(aggA1, aggB1, histAd, histBd, histAs, histBs, bA, bB, WA, WB)


def _tc_post(aggA2, aggB2, histAd, histBd, bA, bB, frame_p, fe_p,
             frfcW, frfcb, fefcW, fefcb):
    return pl.pallas_call(
        _tc_post_body,
        grid=(N_NODE // _RB,),
        in_specs=[_rows_spec(HID), _rows_spec(HID), _rows_spec(HID), _rows_spec(HID),
                  _full_spec(1, HID), _full_spec(1, HID),
                  _rows_spec(HID), _rows_spec(HID),
                  _full_spec(HID, 64), _full_spec(1, 64),
                  _full_spec(HID, 32), _full_spec(1, 32)],
        out_specs=[_rows_spec(64), _rows_spec(32), _rows_spec(HID), _rows_spec(HID)],
        out_shape=[jax.ShapeDtypeStruct((N_NODE, 64), jnp.float32),
                   jax.ShapeDtypeStruct((N_NODE, 32), jnp.float32),
                   jax.ShapeDtypeStruct((N_NODE, HID), jnp.float32),
                   jax.ShapeDtypeStruct((N_NODE, HID), jnp.float32)],
    )(aggA2, aggB2, histAd, histBd, bA, bB, frame_p, fe_p,
      frfcW, frfcb, fefcW, fefcb)


def _prep_edges(idx, fill):
    pad = jnp.full((EPAD - E,), fill, jnp.int32)
    return jnp.concatenate([idx.astype(jnp.int32), pad]).reshape(NSUB, NCHUNK, CHUNK)


@jax.jit
def kernel(frame, fe, edge_e2f, edge_f2e, W_e2f, b_e2f, W_f2e, b_f2e,
           frW, frb, feW, feb, frfcW, frfcb, fefcW, fefcb):
    e_src_g = jnp.stack([_prep_edges(edge_e2f[0], 0),
                         _prep_edges(edge_f2e[0], 0) + N_NODE])
    e_src_h = jnp.stack([_prep_edges(edge_e2f[0], DUMMY),
                         _prep_edges(edge_f2e[0], DUMMY)])
    e_dst = jnp.stack([_prep_edges(edge_e2f[1], DUMMY),
                       _prep_edges(edge_f2e[1], DUMMY)])

    bA = b_e2f.reshape(1, HID)
    bB = b_f2e.reshape(1, HID)

    ones_tab = jnp.ones((2 * N_NODE, HID), jnp.float32)
    zeros_idx = jnp.zeros(e_dst.shape, jnp.int32)
    hist_s = _sc_conv(ones_tab, zeros_idx, e_src_h)
    hist_d = _sc_conv(ones_tab, zeros_idx, e_dst)
    histAs, histBs = hist_s[0, :N_NODE], hist_s[1, :N_NODE]
    histAd, histBd = hist_d[0, :N_NODE], hist_d[1, :N_NODE]
    frame_p, fe_p, featA1, featB1 = _tc_pre(
        frame, fe, histAs, histBs, frW, frb.reshape(1, HID),
        feW, feb.reshape(1, HID), W_e2f, W_f2e)
    agg1 = _sc_conv(jnp.concatenate([featA1, featB1]), e_src_g, e_dst)
    featA2, featB2 = _tc_mid(agg1[0, :N_NODE], agg1[1, :N_NODE], histAd, histBd,
                             histAs, histBs, bA, bB, W_e2f, W_f2e)
    agg2 = _sc_conv(jnp.concatenate([featA2, featB2]), e_src_g, e_dst)
    fr_h, fe_h, h2_frame, h2_fe = _tc_post(
        agg2[0, :N_NODE], agg2[1, :N_NODE], histAd, histBd, bA, bB, frame_p, fe_p,
        frfcW, frfcb.reshape(1, 64), fefcW, fefcb.reshape(1, 32))
    return (fr_h, fe_h, h2_frame, h2_fe)


# speedup vs baseline: 28.2306x; 28.2306x over previous
import functools

import jax
import jax.numpy as jnp
from jax import lax
from jax.experimental import pallas as pl
from jax.experimental.pallas import tpu as pltpu
from jax.experimental.pallas import tpu_sc as plsc

N_NODE = 10000
E = 160000
HID = 128

NSUB = 16
PAD_N = 10240
DUMMY = PAD_N - 1
CHUNK = 128
EPAD = 163840
NCHUNK = EPAD // (NSUB * CHUNK)
ROWS_PER_SUB = PAD_N // NSUB

_sc_mesh = plsc.VectorSubcoreMesh(core_axis_name="c", subcore_axis_name="s")


def _fill_rows(ref, nrows, ncols16, value):
    @pl.loop(0, nrows)
    def _(i):
        for k in range(ncols16):
            ref[i, pl.ds(k * 16, 16)] = jnp.full((16,), value, jnp.float32)


@functools.partial(
    pl.kernel,
    out_type=[jax.ShapeDtypeStruct((2, PAD_N, HID), jnp.float32) for _ in range(2)],
    mesh=_sc_mesh,
    scratch_types=[
        pltpu.VMEM((NCHUNK, CHUNK), jnp.int32),
        pltpu.VMEM((CHUNK, HID), jnp.float32),
        pltpu.VMEM((CHUNK, HID), jnp.float32),
        pltpu.VMEM_SHARED((PAD_N, HID), jnp.float32),
    ],
)
def _sc_hist(e_src, e_dst, hist_s, hist_d, idx, ones_v, zero_v, acc):
    c = lax.axis_index("c")
    s = lax.axis_index("s")
    base = s * ROWS_PER_SUB
    _fill_rows(ones_v, CHUNK, HID // 16, 1.0)
    _fill_rows(zero_v, CHUNK, HID // 16, 0.0)
    for e_idx, hist_out in ((e_src, hist_s), (e_dst, hist_d)):
        for t in range(ROWS_PER_SUB // CHUNK):
            pltpu.sync_copy(zero_v, acc.at[pl.ds(base + t * CHUNK, CHUNK)])
        pltpu.sync_copy(e_idx.at[c, s], idx)
        plsc.subcore_barrier()

        @pl.loop(0, NCHUNK)
        def _(j):
            pltpu.sync_copy(ones_v, acc.at[idx.at[j]], add=True)

        plsc.subcore_barrier()
        pltpu.sync_copy(acc.at[pl.ds(base, ROWS_PER_SUB)],
                        hist_out.at[c, pl.ds(base, ROWS_PER_SUB)])
        plsc.subcore_barrier()


HCHUNK = NCHUNK // 2


@functools.partial(
    pl.kernel,
    out_type=jax.ShapeDtypeStruct((2, PAD_N, HID), jnp.float32),
    mesh=_sc_mesh,
    scratch_types=[
        pltpu.VMEM((HCHUNK, CHUNK), jnp.int32),
        pltpu.VMEM((HCHUNK, CHUNK), jnp.int32),
        pltpu.VMEM((CHUNK, HID), jnp.float32),
        pltpu.VMEM((CHUNK, HID), jnp.float32),
        pltpu.VMEM_SHARED((PAD_N, HID), jnp.float32),
        pltpu.SemaphoreType.DMA,
        pltpu.SemaphoreType.DMA,
    ],
)
def _sc_conv(feat, e_src, e_dst, agg,
             idx_s, idx_d, rows0, rows1, acc, sem0, sem1):
    c = lax.axis_index("c")
    s = lax.axis_index("s")
    base = s * ROWS_PER_SUB
    _fill_rows(rows0, CHUNK, HID // 16, 0.0)
    for t in range(ROWS_PER_SUB // CHUNK):
        pltpu.sync_copy(rows0, acc.at[pl.ds(base + t * CHUNK, CHUNK)])
    plsc.subcore_barrier()

    for h in range(2):
        pltpu.sync_copy(e_src.at[c, s, pl.ds(h * HCHUNK, HCHUNK)], idx_s)
        pltpu.sync_copy(e_dst.at[c, s, pl.ds(h * HCHUNK, HCHUNK)], idx_d)
        pltpu.async_copy(feat.at[idx_s.at[0]], rows0, sem0)

        @pl.loop(0, HCHUNK // 2 - 1)
        def _(t):
            j0 = 2 * t
            pltpu.async_copy(feat.at[idx_s.at[j0 + 1]], rows1, sem1)
            pltpu.make_async_copy(feat.at[idx_s.at[j0]], rows0, sem0).wait()
            pltpu.sync_copy(rows0, acc.at[idx_d.at[j0]], add=True)
            pltpu.async_copy(feat.at[idx_s.at[j0 + 2]], rows0, sem0)
            pltpu.make_async_copy(feat.at[idx_s.at[j0 + 1]], rows1, sem1).wait()
            pltpu.sync_copy(rows1, acc.at[idx_d.at[j0 + 1]], add=True)

        jl = HCHUNK - 2
        pltpu.async_copy(feat.at[idx_s.at[jl + 1]], rows1, sem1)
        pltpu.make_async_copy(feat.at[idx_s.at[jl]], rows0, sem0).wait()
        pltpu.sync_copy(rows0, acc.at[idx_d.at[jl]], add=True)
        pltpu.make_async_copy(feat.at[idx_s.at[jl + 1]], rows1, sem1).wait()
        pltpu.sync_copy(rows1, acc.at[idx_d.at[jl + 1]], add=True)

    plsc.subcore_barrier()
    pltpu.sync_copy(acc.at[pl.ds(base, ROWS_PER_SUB)], agg.at[c, pl.ds(base, ROWS_PER_SUB)])


_RB = 1000


def _norm_col(hist_blk):
    return lax.rsqrt(jnp.maximum(hist_blk[:, 0:1], 1.0))


def _dot(a, b):
    return jax.lax.dot_general(a, b, (((1,), (0,)), ((), ())),
                               preferred_element_type=jnp.float32,
                               precision=jax.lax.Precision.HIGHEST)


def _tc_pre_body(frame_r, fe_r, histAs_r, histBs_r, frW_r, frb_r, feW_r, feb_r,
                 WA_r, WB_r, frame_p_r, fe_p_r, featA1_r, featB1_r):
    frame_p = _dot(frame_r[...], frW_r[...]) + frb_r[...]
    fe_p = _dot(fe_r[...], feW_r[...]) + feb_r[...]
    frame_p_r[...] = frame_p
    fe_p_r[...] = fe_p
    featA1_r[...] = _dot(fe_p * _norm_col(histAs_r[...]), WA_r[...])
    featB1_r[...] = _dot(frame_p * _norm_col(histBs_r[...]), WB_r[...])


def _tc_mid_body(aggA1_r, aggB1_r, histAd_r, histBd_r, histAs_r, histBs_r,
                 bA_r, bB_r, WA_r, WB_r, featA2_r, featB2_r):
    h_frame = jnp.maximum(aggA1_r[...] * _norm_col(histAd_r[...]) + bA_r[...], 0.0)
    h_fe = jnp.maximum(aggB1_r[...] * _norm_col(histBd_r[...]) + bB_r[...], 0.0)
    featA2_r[...] = _dot(h_fe * _norm_col(histAs_r[...]), WA_r[...])
    featB2_r[...] = _dot(h_frame * _norm_col(histBs_r[...]), WB_r[...])


def _tc_post_body(aggA2_r, aggB2_r, histAd_r, histBd_r, bA_r, bB_r,
                  frame_p_r, fe_p_r, frfcW_r, frfcb_r, fefcW_r, fefcb_r,
                  fr_h_r, fe_h_r, h2_frame_r, h2_fe_r):
    h2_frame = aggA2_r[...] * _norm_col(histAd_r[...]) + bA_r[...] + frame_p_r[...]
    h2_fe = aggB2_r[...] * _norm_col(histBd_r[...]) + bB_r[...] + fe_p_r[...]
    h2_frame_r[...] = h2_frame
    h2_fe_r[...] = h2_fe
    fr_h_r[...] = _dot(h2_frame, frfcW_r[...]) + frfcb_r[...]
    fe_h_r[...] = _dot(h2_fe, fefcW_r[...]) + fefcb_r[...]


def _rows_spec(cols):
    return pl.BlockSpec((_RB, cols), lambda i: (i, 0))


def _full_spec(r, cols):
    return pl.BlockSpec((r, cols), lambda i: (0, 0))


def _tc_pre(frame, fe, histAs, histBs, frW, frb, feW, feb, WA, WB):
    return pl.pallas_call(
        _tc_pre_body,
        grid=(N_NODE // _RB,),
        in_specs=[_rows_spec(64), _rows_spec(32), _rows_spec(HID), _rows_spec(HID),
                  _full_spec(64, HID), _full_spec(1, HID),
                  _full_spec(32, HID), _full_spec(1, HID),
                  _full_spec(HID, HID), _full_spec(HID, HID)],
        out_specs=[_rows_spec(HID)] * 4,
        out_shape=[jax.ShapeDtypeStruct((N_NODE, HID), jnp.float32)] * 4,
    )(frame, fe, histAs, histBs, frW, frb, feW, feb, WA, WB)


def _tc_mid(aggA1, aggB1, histAd, histBd, histAs, histBs, bA, bB, WA, WB):
    return pl.pallas_call(
        _tc_mid_body,
        grid=(N_NODE // _RB,),
        in_specs=[_rows_spec(HID), _rows_spec(HID),
                  _rows_spec(HID), _rows_spec(HID), _rows_spec(HID), _rows_spec(HID),
                  _full_spec(1, HID), _full_spec(1, HID),
                  _full_spec(HID, HID), _full_spec(HID, HID)],
        out_specs=[_rows_spec(HID)] * 2,
        out_shape=[jax.ShapeDtypeStruct((N_NODE, HID), jnp.float32)] * 2,
    )(aggA1, aggB1, histAd, histBd, histAs, histBs, bA, bB, WA, WB)


def _tc_post(aggA2, aggB2, histAd, histBd, bA, bB, frame_p, fe_p,
             frfcW, frfcb, fefcW, fefcb):
    return pl.pallas_call(
        _tc_post_body,
        grid=(N_NODE // _RB,),
        in_specs=[_rows_spec(HID), _rows_spec(HID), _rows_spec(HID), _rows_spec(HID),
                  _full_spec(1, HID), _full_spec(1, HID),
                  _rows_spec(HID), _rows_spec(HID),
                  _full_spec(HID, 64), _full_spec(1, 64),
                  _full_spec(HID, 32), _full_spec(1, 32)],
        out_specs=[_rows_spec(64), _rows_spec(32), _rows_spec(HID), _rows_spec(HID)],
        out_shape=[jax.ShapeDtypeStruct((N_NODE, 64), jnp.float32),
                   jax.ShapeDtypeStruct((N_NODE, 32), jnp.float32),
                   jax.ShapeDtypeStruct((N_NODE, HID), jnp.float32),
                   jax.ShapeDtypeStruct((N_NODE, HID), jnp.float32)],
    )(aggA2, aggB2, histAd, histBd, bA, bB, frame_p, fe_p,
      frfcW, frfcb, fefcW, fefcb)


def _prep_edges(idx, fill):
    pad = jnp.full((EPAD - E,), fill, jnp.int32)
    return jnp.concatenate([idx.astype(jnp.int32), pad]).reshape(NSUB, NCHUNK, CHUNK)


@jax.jit
def kernel(frame, fe, edge_e2f, edge_f2e, W_e2f, b_e2f, W_f2e, b_f2e,
           frW, frb, feW, feb, frfcW, frfcb, fefcW, fefcb):
    e_src_g = jnp.stack([_prep_edges(edge_e2f[0], 0),
                         _prep_edges(edge_f2e[0], 0) + N_NODE])
    e_src_h = jnp.stack([_prep_edges(edge_e2f[0], DUMMY),
                         _prep_edges(edge_f2e[0], DUMMY)])
    e_dst = jnp.stack([_prep_edges(edge_e2f[1], DUMMY),
                       _prep_edges(edge_f2e[1], DUMMY)])

    bA = b_e2f.reshape(1, HID)
    bB = b_f2e.reshape(1, HID)

    hist_s, hist_d = _sc_hist(e_src_h, e_dst)
    histAs, histBs = hist_s[0, :N_NODE], hist_s[1, :N_NODE]
    histAd, histBd = hist_d[0, :N_NODE], hist_d[1, :N_NODE]
    frame_p, fe_p, featA1, featB1 = _tc_pre(
        frame, fe, histAs, histBs, frW, frb.reshape(1, HID),
        feW, feb.reshape(1, HID), W_e2f, W_f2e)
    agg1 = _sc_conv(jnp.concatenate([featA1, featB1]), e_src_g, e_dst)
    featA2, featB2 = _tc_mid(agg1[0, :N_NODE], agg1[1, :N_NODE], histAd, histBd,
                             histAs, histBs, bA, bB, W_e2f, W_f2e)
    agg2 = _sc_conv(jnp.concatenate([featA2, featB2]), e_src_g, e_dst)
    fr_h, fe_h, h2_frame, h2_fe = _tc_post(
        agg2[0, :N_NODE], agg2[1, :N_NODE], histAd, histBd, bA, bB, frame_p, fe_p,
        frfcW, frfcb.reshape(1, 64), fefcW, fefcb.reshape(1, 32))
    return (fr_h, fe_h, h2_frame, h2_fe)


# speedup vs baseline: 28.3017x; 1.0025x over previous
import functools

import jax
import jax.numpy as jnp
from jax import lax
from jax.experimental import pallas as pl
from jax.experimental.pallas import tpu as pltpu
from jax.experimental.pallas import tpu_sc as plsc

N_NODE = 10000
E = 160000
HID = 128

NSUB = 16
PAD_N = 10240
DUMMY = PAD_N - 1
CHUNK = 128
EPAD = 163840
NCHUNK = EPAD // (NSUB * CHUNK)
ROWS_PER_SUB = PAD_N // NSUB

_sc_mesh = plsc.VectorSubcoreMesh(core_axis_name="c", subcore_axis_name="s")


def _fill_rows(ref, nrows, ncols16, value):
    @pl.loop(0, nrows)
    def _(i):
        for k in range(ncols16):
            ref[i, pl.ds(k * 16, 16)] = jnp.full((16,), value, jnp.float32)


@functools.partial(
    pl.kernel,
    out_type=[jax.ShapeDtypeStruct((2, PAD_N, HID), jnp.float32) for _ in range(2)],
    mesh=_sc_mesh,
    scratch_types=[
        pltpu.VMEM((NCHUNK, CHUNK), jnp.int32),
        pltpu.VMEM((CHUNK, HID), jnp.float32),
        pltpu.VMEM((CHUNK, HID), jnp.float32),
        pltpu.VMEM_SHARED((PAD_N, HID), jnp.float32),
    ],
)
def _sc_hist(e_src, e_dst, hist_s, hist_d, idx, ones_v, zero_v, acc):
    c = lax.axis_index("c")
    s = lax.axis_index("s")
    base = s * ROWS_PER_SUB
    _fill_rows(ones_v, CHUNK, HID // 16, 1.0)
    _fill_rows(zero_v, CHUNK, HID // 16, 0.0)
    for e_idx, hist_out in ((e_src, hist_s), (e_dst, hist_d)):
        for t in range(ROWS_PER_SUB // CHUNK):
            pltpu.sync_copy(zero_v, acc.at[pl.ds(base + t * CHUNK, CHUNK)])
        pltpu.sync_copy(e_idx.at[c, s], idx)
        plsc.subcore_barrier()

        @pl.loop(0, NCHUNK)
        def _(j):
            pltpu.sync_copy(ones_v, acc.at[idx.at[j]], add=True)

        plsc.subcore_barrier()
        pltpu.sync_copy(acc.at[pl.ds(base, ROWS_PER_SUB)],
                        hist_out.at[c, pl.ds(base, ROWS_PER_SUB)])
        plsc.subcore_barrier()


HCHUNK = NCHUNK // 2


@functools.partial(
    pl.kernel,
    out_type=jax.ShapeDtypeStruct((2, PAD_N, HID), jnp.float32),
    mesh=_sc_mesh,
    scratch_types=[
        pltpu.VMEM((HCHUNK, CHUNK), jnp.int32),
        pltpu.VMEM((HCHUNK, CHUNK), jnp.int32),
        pltpu.VMEM((CHUNK, HID), jnp.float32),
        pltpu.VMEM((CHUNK, HID), jnp.float32),
        pltpu.VMEM_SHARED((PAD_N, HID), jnp.float32),
        pltpu.SemaphoreType.DMA,
        pltpu.SemaphoreType.DMA,
    ],
)
def _sc_conv(feat, e_src, e_dst, agg,
             idx_s, idx_d, rows0, rows1, acc, sem0, sem1):
    c = lax.axis_index("c")
    s = lax.axis_index("s")
    base = s * ROWS_PER_SUB
    _fill_rows(rows0, CHUNK, HID // 16, 0.0)
    for t in range(ROWS_PER_SUB // CHUNK):
        pltpu.sync_copy(rows0, acc.at[pl.ds(base + t * CHUNK, CHUNK)])
    plsc.subcore_barrier()

    for h in range(2):
        pltpu.sync_copy(e_src.at[c, s, pl.ds(h * HCHUNK, HCHUNK)], idx_s)
        pltpu.sync_copy(e_dst.at[c, s, pl.ds(h * HCHUNK, HCHUNK)], idx_d)
        pltpu.async_copy(feat.at[idx_s.at[0]], rows0, sem0)

        @pl.loop(0, HCHUNK // 2 - 1)
        def _(t):
            j0 = 2 * t
            pltpu.async_copy(feat.at[idx_s.at[j0 + 1]], rows1, sem1)
            pltpu.make_async_copy(feat.at[idx_s.at[j0]], rows0, sem0).wait()
            pltpu.sync_copy(rows0, acc.at[idx_d.at[j0]], add=True)
            pltpu.async_copy(feat.at[idx_s.at[j0 + 2]], rows0, sem0)
            pltpu.make_async_copy(feat.at[idx_s.at[j0 + 1]], rows1, sem1).wait()
            pltpu.sync_copy(rows1, acc.at[idx_d.at[j0 + 1]], add=True)

        jl = HCHUNK - 2
        pltpu.async_copy(feat.at[idx_s.at[jl + 1]], rows1, sem1)
        pltpu.make_async_copy(feat.at[idx_s.at[jl]], rows0, sem0).wait()
        pltpu.sync_copy(rows0, acc.at[idx_d.at[jl]], add=True)
        pltpu.make_async_copy(feat.at[idx_s.at[jl + 1]], rows1, sem1).wait()
        pltpu.sync_copy(rows1, acc.at[idx_d.at[jl + 1]], add=True)

    plsc.subcore_barrier()
    pltpu.sync_copy(acc.at[pl.ds(base, ROWS_PER_SUB)], agg.at[c, pl.ds(base, ROWS_PER_SUB)])


_RB = PAD_N // 10
_RBP = 1000


def _norm_col(hist_blk):
    return lax.rsqrt(jnp.maximum(hist_blk[:, 0:1], 1.0))


def _dot(a, b):
    return jax.lax.dot_general(a, b, (((1,), (0,)), ((), ())),
                               preferred_element_type=jnp.float32,
                               precision=jax.lax.Precision.HIGHEST)


def _tc_pre_body(x_r, hs_r, Win_r, bin_r, Wrel_r, p_r, feat_r):
    p = _dot(x_r[0], Win_r[0]) + bin_r[0]
    p_r[0] = p
    feat_r[0] = _dot(p * _norm_col(hs_r[0]), Wrel_r[0])


def _tc_mid_body(agg_r, hd_r, hs_r, brel_r, Wrel_r, feat_r):
    h = jnp.maximum(agg_r[0] * _norm_col(hd_r[0]) + brel_r[0], 0.0)
    feat_r[0] = _dot(h * _norm_col(hs_r[0]), Wrel_r[0])


def _tc_post_body(aggA_r, aggB_r, hdA_r, hdB_r, bA_r, bB_r, fep_r, frp_r,
                  frfcW_r, frfcb_r, fefcW_r, fefcb_r,
                  fr_h_r, fe_h_r, h2_frame_r, h2_fe_r):
    h2_frame = aggA_r[0] * _norm_col(hdA_r[0]) + bA_r[0] + frp_r[0]
    h2_fe = aggB_r[0] * _norm_col(hdB_r[0]) + bB_r[0] + fep_r[0]
    h2_frame_r[...] = h2_frame
    h2_fe_r[...] = h2_fe
    fr_h_r[...] = _dot(h2_frame, frfcW_r[...]) + frfcb_r[...]
    fe_h_r[...] = _dot(h2_fe, fefcW_r[...]) + fefcb_r[...]


def _tc_pre(x_in, hist_s, W_in, b_in, W_rel):
    return pl.pallas_call(
        _tc_pre_body,
        grid=(2, PAD_N // _RB),
        in_specs=[
            pl.BlockSpec((1, _RB, 64), lambda r, i: (r, i, 0)),
            pl.BlockSpec((1, _RB, HID), lambda r, i: (r, i, 0)),
            pl.BlockSpec((1, 64, HID), lambda r, i: (r, 0, 0)),
            pl.BlockSpec((1, 1, HID), lambda r, i: (r, 0, 0)),
            pl.BlockSpec((1, HID, HID), lambda r, i: (r, 0, 0)),
        ],
        out_specs=[pl.BlockSpec((1, _RB, HID), lambda r, i: (r, i, 0))] * 2,
        out_shape=[jax.ShapeDtypeStruct((2, PAD_N, HID), jnp.float32)] * 2,
    )(x_in, hist_s, W_in, b_in, W_rel)


def _tc_mid(agg1, hist_d, hist_s, b_rel, W_rel):
    return pl.pallas_call(
        _tc_mid_body,
        grid=(2, PAD_N // _RB),
        in_specs=[
            pl.BlockSpec((1, _RB, HID), lambda r, i: (1 - r, i, 0)),
            pl.BlockSpec((1, _RB, HID), lambda r, i: (1 - r, i, 0)),
            pl.BlockSpec((1, _RB, HID), lambda r, i: (r, i, 0)),
            pl.BlockSpec((1, 1, HID), lambda r, i: (1 - r, 0, 0)),
            pl.BlockSpec((1, HID, HID), lambda r, i: (r, 0, 0)),
        ],
        out_specs=pl.BlockSpec((1, _RB, HID), lambda r, i: (r, i, 0)),
        out_shape=jax.ShapeDtypeStruct((2, PAD_N, HID), jnp.float32),
    )(agg1, hist_d, hist_s, b_rel, W_rel)


def _tc_post(agg2, hist_d, b_rel, p, frfcW, frfcb, fefcW, fefcb):
    rows = pl.BlockSpec((1, _RBP, HID), lambda i: (0, i, 0))
    rows1 = pl.BlockSpec((1, _RBP, HID), lambda i: (1, i, 0))
    return pl.pallas_call(
        _tc_post_body,
        grid=(N_NODE // _RBP,),
        in_specs=[rows, rows1, rows, rows1,
                  pl.BlockSpec((1, 1, HID), lambda i: (0, 0, 0)),
                  pl.BlockSpec((1, 1, HID), lambda i: (1, 0, 0)),
                  rows, rows1,
                  pl.BlockSpec((HID, 64), lambda i: (0, 0)),
                  pl.BlockSpec((1, 64), lambda i: (0, 0)),
                  pl.BlockSpec((HID, 32), lambda i: (0, 0)),
                  pl.BlockSpec((1, 32), lambda i: (0, 0))],
        out_specs=[pl.BlockSpec((_RBP, 64), lambda i: (i, 0)),
                   pl.BlockSpec((_RBP, 32), lambda i: (i, 0)),
                   pl.BlockSpec((_RBP, HID), lambda i: (i, 0)),
                   pl.BlockSpec((_RBP, HID), lambda i: (i, 0))],
        out_shape=[jax.ShapeDtypeStruct((N_NODE, 64), jnp.float32),
                   jax.ShapeDtypeStruct((N_NODE, 32), jnp.float32),
                   jax.ShapeDtypeStruct((N_NODE, HID), jnp.float32),
                   jax.ShapeDtypeStruct((N_NODE, HID), jnp.float32)],
    )(agg2, agg2, hist_d, hist_d, b_rel, b_rel, p, p,
      frfcW, frfcb, fefcW, fefcb)


def _prep_edges(idx, fill):
    pad = jnp.full((EPAD - E,), fill, jnp.int32)
    return jnp.concatenate([idx.astype(jnp.int32), pad]).reshape(NSUB, NCHUNK, CHUNK)


@jax.jit
def kernel(frame, fe, edge_e2f, edge_f2e, W_e2f, b_e2f, W_f2e, b_f2e,
           frW, frb, feW, feb, frfcW, frfcb, fefcW, fefcb):
    e_src_g = jnp.stack([_prep_edges(edge_e2f[0], 0),
                         _prep_edges(edge_f2e[0], 0) + PAD_N])
    e_src_h = jnp.stack([_prep_edges(edge_e2f[0], DUMMY),
                         _prep_edges(edge_f2e[0], DUMMY)])
    e_dst = jnp.stack([_prep_edges(edge_e2f[1], DUMMY),
                       _prep_edges(edge_f2e[1], DUMMY)])

    x_in = jnp.stack([jnp.pad(fe, ((0, PAD_N - N_NODE), (0, 32))),
                      jnp.pad(frame, ((0, PAD_N - N_NODE), (0, 0)))])
    W_in = jnp.stack([jnp.pad(feW, ((0, 32), (0, 0))), frW])
    b_in = jnp.stack([feb.reshape(1, HID), frb.reshape(1, HID)])
    b_rel = jnp.stack([b_e2f.reshape(1, HID), b_f2e.reshape(1, HID)])
    W_rel = jnp.stack([W_e2f, W_f2e])

    hist_s, hist_d = _sc_hist(e_src_h, e_dst)
    p, feat1 = _tc_pre(x_in, hist_s, W_in, b_in, W_rel)
    agg1 = _sc_conv(feat1.reshape(2 * PAD_N, HID), e_src_g, e_dst)
    feat2 = _tc_mid(agg1, hist_d, hist_s, b_rel, W_rel)
    agg2 = _sc_conv(feat2.reshape(2 * PAD_N, HID), e_src_g, e_dst)
    fr_h, fe_h, h2_frame, h2_fe = _tc_post(
        agg2, hist_d, b_rel, p,
        frfcW, frfcb.reshape(1, 64), fefcW, fefcb.reshape(1, 32))
    return (fr_h, fe_h, h2_frame, h2_fe)
